# baseline (speedup 1.0000x reference)
import functools

import jax
import jax.numpy as jnp
from jax import lax
from jax.experimental import pallas as pl
from jax.experimental.pallas import tpu as pltpu
from jax.experimental.pallas import tpu_sc as plsc

D = 256
K = 8192
RB = 512
WB = 1024


def _argmin_body(x_ref, w_ref, wsq_ref, idx_ref, min_s, arg_s):
    j = pl.program_id(1)
    nj = pl.num_programs(1)
    xb = x_ref[...]
    wb = w_ref[...]
    mm = lax.dot_general(xb, wb, (((1,), (1,)), ((), ())),
                         preferred_element_type=jnp.float32)
    xsq = jnp.sum(xb * xb, axis=1, keepdims=True)
    dist = (wsq_ref[...] - 2.0 * mm) + xsq
    lmin = jnp.min(dist, axis=1, keepdims=True)
    ids = lax.broadcasted_iota(jnp.int32, (RB, WB), 1)
    larg = jnp.min(jnp.where(dist == lmin, ids, K),
                   axis=1, keepdims=True) + j * WB

    @pl.when(j == 0)
    def _():
        min_s[...] = lmin
        arg_s[...] = larg

    @pl.when(j > 0)
    def _():
        upd = lmin < min_s[...]
        arg_s[...] = jnp.where(upd, larg, arg_s[...])
        min_s[...] = jnp.where(upd, lmin, min_s[...])

    @pl.when(j == nj - 1)
    def _():
        idx_ref[...] = arg_s[...]


def _argmin_indices(flat, weight, wsq):
    n = flat.shape[0]
    grid = (n // RB, K // WB)
    out = pl.pallas_call(
        _argmin_body,
        grid=grid,
        in_specs=[
            pl.BlockSpec((RB, D), lambda i, j: (i, 0)),
            pl.BlockSpec((WB, D), lambda i, j: (j, 0)),
            pl.BlockSpec((1, WB), lambda i, j: (0, j)),
        ],
        out_specs=pl.BlockSpec((RB, 1), lambda i, j: (i, 0)),
        out_shape=jax.ShapeDtypeStruct((n, 1), jnp.int32),
        scratch_shapes=[
            pltpu.VMEM((RB, 1), jnp.float32),
            pltpu.VMEM((RB, 1), jnp.int32),
        ],
        compiler_params=pltpu.CompilerParams(
            dimension_semantics=("parallel", "arbitrary"),
        ),
    )(flat, weight, wsq)
    return out.reshape(n)


def _make_sc_gather(n):
    info = plsc.get_sparse_core_info()
    nw = info.num_cores * info.num_subcores
    b_per_w = n // nw
    chunk = 128
    nchunks = b_per_w // chunk
    mesh = plsc.VectorSubcoreMesh(core_axis_name="c", subcore_axis_name="s")

    @functools.partial(
        pl.kernel,
        out_type=jax.ShapeDtypeStruct((n, D), jnp.float32),
        mesh=mesh,
        scratch_types=[
            pltpu.VMEM((b_per_w,), jnp.int32),
            pltpu.VMEM((2, chunk, D), jnp.float32),
            pltpu.SemaphoreType.DMA,
            pltpu.SemaphoreType.DMA,
        ],
    )
    def gather(table_hbm, idx_hbm, out_hbm, idx_v, rows_v, sem0, sem1):
        wid = lax.axis_index("s") * info.num_cores + lax.axis_index("c")
        base = wid * b_per_w
        pltpu.sync_copy(idx_hbm.at[pl.ds(base, b_per_w)], idx_v)
        sems = (sem0, sem1)
        copies = [None, None]
        for c in range(nchunks):
            s = c % 2
            copies[s] = pltpu.async_copy(
                table_hbm.at[idx_v.at[pl.ds(c * chunk, chunk)]],
                rows_v.at[s], sems[s])
            if c > 0:
                p = (c - 1) % 2
                copies[p].wait()
                pltpu.sync_copy(
                    rows_v.at[p],
                    out_hbm.at[pl.ds(base + (c - 1) * chunk, chunk)])
        copies[(nchunks - 1) % 2].wait()
        pltpu.sync_copy(
            rows_v.at[(nchunks - 1) % 2],
            out_hbm.at[pl.ds(base + (nchunks - 1) * chunk, chunk)])

    return gather


def kernel(x, weight):
    b, t, d = x.shape
    n = b * t
    flat = x.reshape(n, d)
    wsq = jnp.sum(jnp.power(weight, 2), axis=1).reshape(1, K)
    idx_flat = _argmin_indices(flat, weight, wsq)
    embed_flat = _make_sc_gather(n)(weight, idx_flat)
    embed = embed_flat.reshape(b, t, d)
    qx = x + (embed - x)
    idx = idx_flat.reshape(b, t)
    return (embed, qx, idx)
